# baseline (speedup 1.0000x reference)
import numpy as np
import jax
import jax.numpy as jnp
from jax.experimental import pallas as pl
from jax.experimental.pallas import tpu as pltpu

_ALPHA = 4
_LANES = 128


def _make_body(n_steps):
    def _body(*refs):
        x_any, srcs, fast_any, slow, sem = refs[0], refs[1:-3], refs[-3], refs[-2], refs[-1]
        i = pl.program_id(0)

        @pl.when(i == 0)
        def _():
            pltpu.make_async_copy(x_any, fast_any, sem).start()

        for k, s in enumerate(srcs):
            slow[:, k] = s[:, 0]

        @pl.when(i == n_steps - 1)
        def _():
            pltpu.make_async_copy(x_any, fast_any, sem).wait()

    return _body


def kernel(frames):
    temporal_axis = 1 if frames.ndim == 4 else 2
    T = frames.shape[temporal_axis]
    S = T // _ALPHA
    idx = tuple(int(v) for v in np.linspace(0.0, T - 1, S))

    if frames.ndim == 4:
        C, _, H, W = frames.shape
        lead = C
    else:
        B, C, _, H, W = frames.shape
        lead = B * C

    hw = H * W
    rows = hw // _LANES
    x = frames.reshape(lead, T, rows, _LANES)

    def _spec(t):
        return pl.BlockSpec((1, 1, rows, _LANES), lambda i, _t=t: (i, _t, 0, 0))

    fast, slow = pl.pallas_call(
        _make_body(lead),
        grid=(lead,),
        in_specs=[pl.BlockSpec(memory_space=pl.ANY)] + [_spec(t) for t in idx],
        out_specs=[
            pl.BlockSpec(memory_space=pl.ANY),
            pl.BlockSpec((1, S, rows, _LANES), lambda i: (i, 0, 0, 0)),
        ],
        out_shape=[
            jax.ShapeDtypeStruct((lead, T, rows, _LANES), frames.dtype),
            jax.ShapeDtypeStruct((lead, S, rows, _LANES), frames.dtype),
        ],
        scratch_shapes=[pltpu.SemaphoreType.DMA],
    )(*([x] * (S + 1)))

    if frames.ndim == 4:
        slow = slow.reshape(C, S, H, W)
        fast = fast.reshape(C, T, H, W)
    else:
        slow = slow.reshape(B, C, S, H, W)
        fast = fast.reshape(B, C, T, H, W)
    return (slow, fast)


# speedup vs baseline: 9.3222x; 9.3222x over previous
import numpy as np
import jax
import jax.numpy as jnp
from jax.experimental import pallas as pl
from jax.experimental.pallas import tpu as pltpu

_ALPHA = 4
_LANES = 128
_NBUF = 4


def _make_body(lead, T, S, rows, idx):
    def _body(x, fast, slow, buf, sem_in, sem_out):
        def in_copy(i):
            b = i % _NBUF
            return pltpu.make_async_copy(x.at[i], buf.at[b], sem_in.at[b])

        def out_copies(i):
            b = i % _NBUF
            cs = [pltpu.make_async_copy(buf.at[b], fast.at[i], sem_out.at[b])]
            for k, t in enumerate(idx):
                cs.append(pltpu.make_async_copy(
                    buf.at[b, t], slow.at[i, k], sem_out.at[b]))
            return cs

        for i in range(lead):
            if i >= _NBUF:
                for c in out_copies(i - _NBUF):
                    c.wait()
            in_copy(i).start()
            if i >= 1:
                in_copy(i - 1).wait()
                for c in out_copies(i - 1):
                    c.start()
        in_copy(lead - 1).wait()
        for c in out_copies(lead - 1):
            c.start()
        for j in range(max(lead - _NBUF, 0), lead):
            for c in out_copies(j):
                c.wait()

    return _body


def kernel(frames):
    temporal_axis = 1 if frames.ndim == 4 else 2
    T = frames.shape[temporal_axis]
    S = T // _ALPHA
    idx = tuple(int(v) for v in np.linspace(0.0, T - 1, S))

    if frames.ndim == 4:
        C, _, H, W = frames.shape
        lead = C
    else:
        B, C, _, H, W = frames.shape
        lead = B * C

    hw = H * W
    rows = hw // _LANES
    x = frames.reshape(lead, T, rows, _LANES)

    fast, slow = pl.pallas_call(
        _make_body(lead, T, S, rows, idx),
        in_specs=[pl.BlockSpec(memory_space=pl.ANY)],
        out_specs=[
            pl.BlockSpec(memory_space=pl.ANY),
            pl.BlockSpec(memory_space=pl.ANY),
        ],
        out_shape=[
            jax.ShapeDtypeStruct((lead, T, rows, _LANES), frames.dtype),
            jax.ShapeDtypeStruct((lead, S, rows, _LANES), frames.dtype),
        ],
        scratch_shapes=[
            pltpu.VMEM((_NBUF, T, rows, _LANES), jnp.float32),
            pltpu.SemaphoreType.DMA((_NBUF,)),
            pltpu.SemaphoreType.DMA((_NBUF,)),
        ],
    )(x)

    if frames.ndim == 4:
        slow = slow.reshape(C, S, H, W)
        fast = fast.reshape(C, T, H, W)
    else:
        slow = slow.reshape(B, C, S, H, W)
        fast = fast.reshape(B, C, T, H, W)
    return (slow, fast)


# speedup vs baseline: 9.8797x; 1.0598x over previous
import functools
import jax
import jax.numpy as jnp
from jax import lax
from jax.experimental import pallas as pl
from jax.experimental.pallas import tpu as pltpu
from jax.experimental.pallas import tpu_sc as plsc

_ALPHA = 4
_NUM_CORES = 2
_NUM_SUBCORES = 16


def _make_sc_pack(lead, T, S, row):
    n_fast = lead * T
    n_slow = lead * S
    n_workers = _NUM_CORES * _NUM_SUBCORES
    assert n_fast % n_workers == 0 and n_slow % n_workers == 0
    fast_per = n_fast // n_workers
    slow_per = n_slow // n_workers
    n_tot = fast_per + slow_per
    mesh = plsc.VectorSubcoreMesh(core_axis_name="c", subcore_axis_name="s")

    @functools.partial(
        pl.kernel,
        mesh=mesh,
        out_type=[
            jax.ShapeDtypeStruct((n_fast, row), jnp.float32),
            jax.ShapeDtypeStruct((n_slow, row), jnp.float32),
        ],
        scratch_types=[
            pltpu.VMEM((2, row), jnp.float32),
            pltpu.SemaphoreType.DMA((2,)),
            pltpu.SemaphoreType.DMA((2,)),
        ],
    )
    def sc_pack(x_hbm, fast_hbm, slow_hbm, buf, sem_in, sem_out):
        wid = lax.axis_index("s") * _NUM_CORES + lax.axis_index("c")

        def job(m):
            if m < fast_per:
                src = wid + n_workers * m
                return src, fast_hbm, src
            j = m - fast_per
            r = wid * slow_per + j
            i = r // S
            k = r - i * S
            src = i * T + ((T - 1) * k) // (S - 1)
            return src, slow_hbm, r

        in_h = [None] * n_tot
        out_h = [None] * n_tot

        def start_out(m):
            _, dst_ref, dst = job(m)
            b = m % 2
            return pltpu.async_copy(buf.at[b], dst_ref.at[dst], sem_out.at[b])

        for m in range(n_tot):
            b = m % 2
            if m >= 2:
                out_h[m - 2].wait()
            src, _, _ = job(m)
            in_h[m] = pltpu.async_copy(x_hbm.at[src], buf.at[b], sem_in.at[b])
            if m >= 1:
                in_h[m - 1].wait()
                out_h[m - 1] = start_out(m - 1)
        in_h[n_tot - 1].wait()
        out_h[n_tot - 1] = start_out(n_tot - 1)
        out_h[n_tot - 2].wait()
        out_h[n_tot - 1].wait()

    return sc_pack


def kernel(frames):
    temporal_axis = 1 if frames.ndim == 4 else 2
    T = frames.shape[temporal_axis]
    S = T // _ALPHA

    if frames.ndim == 4:
        C, _, H, W = frames.shape
        lead = C
    else:
        B, C, _, H, W = frames.shape
        lead = B * C

    row = H * W
    x = frames.reshape(lead * T, row)
    fast, slow = _make_sc_pack(lead, T, S, row)(x)

    if frames.ndim == 4:
        slow = slow.reshape(C, S, H, W)
        fast = fast.reshape(C, T, H, W)
    else:
        slow = slow.reshape(B, C, S, H, W)
        fast = fast.reshape(B, C, T, H, W)
    return (slow, fast)


# speedup vs baseline: 35.9443x; 3.6382x over previous
import numpy as np
import jax
import jax.numpy as jnp
from jax.experimental import pallas as pl

_ALPHA = 4


def _gather_body(*refs):
    srcs, out = refs[:-1], refs[-1]
    for k, s in enumerate(srcs):
        out[0, 0, k] = s[0, 0, 0]


def _gather_body_4d(*refs):
    srcs, out = refs[:-1], refs[-1]
    for k, s in enumerate(srcs):
        out[0, k] = s[0, 0]


def kernel(frames):
    temporal_axis = 1 if frames.ndim == 4 else 2
    T = frames.shape[temporal_axis]
    S = T // _ALPHA
    idx = tuple(int(v) for v in np.linspace(0.0, T - 1, S))

    if frames.ndim == 4:
        C, _, H, W = frames.shape

        def _spec(t):
            return pl.BlockSpec((1, 1, H, W), lambda c, _t=t: (c, _t, 0, 0))

        slow = pl.pallas_call(
            _gather_body_4d,
            grid=(C,),
            in_specs=[_spec(t) for t in idx],
            out_specs=pl.BlockSpec((1, S, H, W), lambda c: (c, 0, 0, 0)),
            out_shape=jax.ShapeDtypeStruct((C, S, H, W), frames.dtype),
        )(*([frames] * S))
        return (slow, frames)

    B, C, _, H, W = frames.shape

    def _spec5(t):
        return pl.BlockSpec((1, 1, 1, H, W), lambda b, c, _t=t: (b, c, _t, 0, 0))

    slow = pl.pallas_call(
        _gather_body,
        grid=(B, C),
        in_specs=[_spec5(t) for t in idx],
        out_specs=pl.BlockSpec((1, 1, S, H, W), lambda b, c: (b, c, 0, 0, 0)),
        out_shape=jax.ShapeDtypeStruct((B, C, S, H, W), frames.dtype),
    )(*([frames] * S))
    return (slow, frames)
